# baseline (speedup 1.0000x reference)
import functools
import jax
import jax.numpy as jnp
from jax import lax
from jax.experimental import pallas as pl
from jax.experimental.pallas import tpu as pltpu
from jax.experimental.pallas import tpu_sc as plsc

_N, _P, _D = 4, 4096, 3
_NW = 32
_SPW = (_N * _P) // _NW
_WPB = _P // _SPW
_L = 16

_GDN = lax.GatherDimensionNumbers(
    offset_dims=(), collapsed_slice_dims=(0,), start_index_map=(0,))


def _lane_bcast(v, j):
    jv = jnp.full((_L, 1), j, jnp.int32)
    return lax.gather(v, jv, _GDN, (1,),
                      mode=lax.GatherScatterMode.PROMISE_IN_BOUNDS)


def _lane_min(v):
    iota = lax.iota(jnp.int32, _L)
    for k in (8, 4, 2, 1):
        perm = (iota ^ k).reshape(_L, 1)
        v = jnp.minimum(
            v, lax.gather(v, perm, _GDN, (1,),
                          mode=lax.GatherScatterMode.PROMISE_IN_BOUNDS))
    return v


def _bf16_round(v):
    c = v * jnp.float32(65537.0)
    return c - (c - v)


def _chamfer_sc(src_hbm, tgt_hbm, out_hbm,
                tx_v, ty_v, tz_v, y2_v,
                sx_v, sy_v, sz_v, x2_v, res_v, sem):
    wid = lax.axis_index("s") * 2 + lax.axis_index("c")
    b = wid // _WPB
    s0 = (wid % _WPB) * _SPW

    tb = b * (3 * _P)
    pltpu.sync_copy(tgt_hbm.at[pl.ds(tb, _P)], tx_v)
    pltpu.sync_copy(tgt_hbm.at[pl.ds(tb + _P, _P)], ty_v)
    pltpu.sync_copy(tgt_hbm.at[pl.ds(tb + 2 * _P, _P)], tz_v)
    pltpu.sync_copy(src_hbm.at[pl.ds(tb + s0, _SPW)], sx_v)
    pltpu.sync_copy(src_hbm.at[pl.ds(tb + _P + s0, _SPW)], sy_v)
    pltpu.sync_copy(src_hbm.at[pl.ds(tb + 2 * _P + s0, _SPW)], sz_v)

    def stage_tgt(q, _):
        sl = pl.ds(q * _L, _L)
        tx, ty, tz = tx_v[sl], ty_v[sl], tz_v[sl]
        y2_v[sl] = tx * tx + ty * ty + tz * tz
        tx_v[sl] = _bf16_round(tx)
        ty_v[sl] = _bf16_round(ty)
        tz_v[sl] = _bf16_round(tz)
        return _

    lax.fori_loop(0, _P // _L, stage_tgt, 0)

    def stage_src(i, _):
        sl = pl.ds(i * _L, _L)
        sx, sy, sz = sx_v[sl], sy_v[sl], sz_v[sl]
        x2_v[sl] = sx * sx + sy * sy + sz * sz
        sx_v[sl] = _bf16_round(sx)
        sy_v[sl] = _bf16_round(sy)
        sz_v[sl] = _bf16_round(sz)
        return _

    lax.fori_loop(0, _SPW // _L, stage_src, 0)

    def per_chunk(i0, total):
        sl = pl.ds(i0 * _L, _L)
        sxv, syv, szv, x2v = sx_v[sl], sy_v[sl], sz_v[sl], x2_v[sl]

        def per_point(j, tot):
            sx = _lane_bcast(sxv, j)
            sy = _lane_bcast(syv, j)
            sz = _lane_bcast(szv, j)
            x2 = _lane_bcast(x2v, j)

            def per_tgt(q, acc):
                qsl = pl.ds(q * _L, _L)
                d = ((x2 + y2_v[qsl])
                     - 2.0 * (sx * tx_v[qsl] + sy * ty_v[qsl]
                              + sz * tz_v[qsl]))
                return jnp.minimum(acc, d)

            acc = lax.fori_loop(0, _P // _L, per_tgt,
                                jnp.full((_L,), jnp.inf, jnp.float32))
            return tot + _lane_min(acc)

        return lax.fori_loop(0, _L, per_point, total)

    total = lax.fori_loop(0, _SPW // _L, per_chunk,
                          jnp.zeros((_L,), jnp.float32))

    res_v[...] = jnp.where(lax.iota(jnp.int32, _L) == 0, total, 0.0)
    pltpu.sync_copy(res_v, out_hbm.at[wid])


def kernel(source_cloud, target_cloud):
    src_t = source_cloud.transpose(0, 2, 1).reshape(-1)
    tgt_t = target_cloud.transpose(0, 2, 1).reshape(-1)
    mesh = plsc.VectorSubcoreMesh(core_axis_name="c", subcore_axis_name="s")
    k = functools.partial(
        pl.kernel, mesh=mesh,
        out_type=jax.ShapeDtypeStruct((_NW, _L), jnp.float32),
        scratch_types=[
            pltpu.VMEM((_P,), jnp.float32),
            pltpu.VMEM((_P,), jnp.float32),
            pltpu.VMEM((_P,), jnp.float32),
            pltpu.VMEM((_P,), jnp.float32),
            pltpu.VMEM((_SPW,), jnp.float32),
            pltpu.VMEM((_SPW,), jnp.float32),
            pltpu.VMEM((_SPW,), jnp.float32),
            pltpu.VMEM((_SPW,), jnp.float32),
            pltpu.VMEM((_L,), jnp.float32),
            pltpu.SemaphoreType.DMA,
        ],
    )(_chamfer_sc)
    partial_sums = k(src_t, tgt_t)
    return jnp.sum(partial_sums) * (1.0 / _N)


# speedup vs baseline: 10.5585x; 10.5585x over previous
import jax
import jax.numpy as jnp
from jax.experimental import pallas as pl
from jax.experimental.pallas import tpu as pltpu

_N, _P, _D = 4, 4096, 3
_QC = 1024
_NQ = _P // _QC


def _chamfer_kernel(src_ref, tgt_ref, out_ref):
    b = pl.program_id(0)

    St = src_ref[...]
    T = tgt_ref[0]

    x2 = jnp.sum(St * St, axis=0, keepdims=True)
    y2 = jnp.sum(T * T, axis=1, keepdims=True)

    y2_hi = y2.astype(jnp.bfloat16).astype(jnp.float32)
    y2_lo = y2 - y2_hi
    L = jnp.concatenate([T, y2_hi, y2_lo], axis=1)
    ones_p = jnp.ones((1, _P), jnp.float32)
    R = jnp.concatenate([-2.0 * St, ones_p, ones_p],
                        axis=0)

    m = None
    for j in range(_NQ):
        d = jax.lax.dot_general(
            L[j * _QC:(j + 1) * _QC], R, (((1,), (0,)), ((), ())),
            preferred_element_type=jnp.float32,
        )
        mj = jnp.min(d, axis=0, keepdims=True)
        m = mj if m is None else jnp.minimum(m, mj)

    s = jnp.sum(m + x2, keepdims=True) * (1.0 / _N)

    @pl.when(b == 0)
    def _():
        out_ref[...] = jnp.zeros_like(out_ref)

    out_ref[...] += s


def kernel(source_cloud, target_cloud):
    src_t = source_cloud.reshape(_N * _P, _D).T
    out = pl.pallas_call(
        _chamfer_kernel,
        grid=(_N,),
        in_specs=[
            pl.BlockSpec((_D, _P), lambda b: (0, b)),
            pl.BlockSpec((1, _P, _D), lambda b: (b, 0, 0)),
        ],
        out_specs=pl.BlockSpec((1, 1), lambda b: (0, 0)),
        out_shape=jax.ShapeDtypeStruct((1, 1), jnp.float32),
    )(src_t, target_cloud)
    return out[0, 0]
